# Initial kernel scaffold; baseline (speedup 1.0000x reference)
import jax
import jax.numpy as jnp
from jax.experimental import pallas as pl


def kernel(node_features, edge_index, edge_weight, W1, b1, W2, b2):
    raise NotImplementedError("write your pallas kernel here")


# speedup vs baseline: 2.8677x; 2.8677x over previous
import functools

import jax
import jax.numpy as jnp
from jax import lax
from jax.experimental import pallas as pl
from jax.experimental.pallas import tpu as pltpu
from jax.experimental.pallas import tpu_sc as plsc

N = 10000
N_PAD = 10240
E = 320000
E_PAD = 327680
CHUNK = 128
D1 = 128
C = 40
D2 = 48
NC = 2
NS = 16
NW = NC * NS
EDGES_PER_W = E_PAD // NW
CHUNKS_PER_W = EDGES_PER_W // CHUNK
ROWS_PER_TILE = N_PAD // NS


def _linear_block(x_ref, w_ref, b_ref, o_ref):
    o_ref[...] = lax.dot_general(
        x_ref[...], w_ref[...], (((1,), (1,)), ((), ())),
        preferred_element_type=jnp.float32) + b_ref[...]


def _linear(x, w, b, bm=512):
    m, k = x.shape
    dout = w.shape[0]
    return pl.pallas_call(
        _linear_block,
        grid=(m // bm,),
        in_specs=[
            pl.BlockSpec((bm, k), lambda i: (i, 0)),
            pl.BlockSpec((dout, k), lambda i: (0, 0)),
            pl.BlockSpec((1, dout), lambda i: (0, 0)),
        ],
        out_specs=pl.BlockSpec((bm, dout), lambda i: (i, 0)),
        out_shape=jax.ShapeDtypeStruct((m, dout), jnp.float32),
    )(x, w, b.reshape(1, dout))


def _fused_linear_block(pa_ref, pb_ref, w_ref, b_ref, o_ref):
    x = jnp.maximum(pa_ref[...] + pb_ref[...], 0.0)
    o_ref[...] = lax.dot_general(
        x, w_ref[...], (((1,), (1,)), ((), ())),
        preferred_element_type=jnp.float32) + b_ref[...]


def _fused_linear(pa, pb, w, b, bm=512):
    m, k = pa.shape
    dout = w.shape[0]
    return pl.pallas_call(
        _fused_linear_block,
        grid=(m // bm,),
        in_specs=[
            pl.BlockSpec((bm, k), lambda i: (i, 0)),
            pl.BlockSpec((bm, k), lambda i: (i, 0)),
            pl.BlockSpec((dout, k), lambda i: (0, 0)),
            pl.BlockSpec((1, dout), lambda i: (0, 0)),
        ],
        out_specs=pl.BlockSpec((bm, dout), lambda i: (i, 0)),
        out_shape=jax.ShapeDtypeStruct((m, dout), jnp.float32),
    )(pa, pb, w, b.reshape(1, dout))


def _logsoftmax_block(pa_ref, pb_ref, o_ref):
    z = pa_ref[...] + pb_ref[...]
    colid = lax.broadcasted_iota(jnp.int32, z.shape, 1)
    valid = colid < C
    zm = jnp.where(valid, z, -jnp.inf)
    mx = jnp.max(zm, axis=1, keepdims=True)
    ex = jnp.where(valid, jnp.exp(z - mx), 0.0)
    s = jnp.sum(ex, axis=1, keepdims=True)
    o_ref[...] = (z - mx - jnp.log(s))[:, :C]


def _logsoftmax(pa, pb, bm=512):
    m, k = pa.shape
    return pl.pallas_call(
        _logsoftmax_block,
        grid=(m // bm,),
        in_specs=[
            pl.BlockSpec((bm, k), lambda i: (i, 0)),
            pl.BlockSpec((bm, k), lambda i: (i, 0)),
        ],
        out_specs=pl.BlockSpec((bm, C), lambda i: (i, 0)),
        out_shape=jax.ShapeDtypeStruct((m, C), jnp.float32),
    )(pa, pb)


def _make_agg(d):
    nvec = d // 16
    mesh = plsc.VectorSubcoreMesh(core_axis_name="c", subcore_axis_name="s")

    def body(sup_hbm, col_hbm, row_hbm, w_hbm, zero_hbm, out_hbm,
             col_v, row_v, w_v, rows_v, acc_sh, sem):
        cid = lax.axis_index("c")
        sid = lax.axis_index("s")
        wid = sid * NC + cid
        slab = pl.ds(sid * ROWS_PER_TILE, ROWS_PER_TILE)
        pltpu.sync_copy(zero_hbm.at[slab], acc_sh.at[slab])
        plsc.subcore_barrier()

        def chunk_body(g, carry):
            base = wid * EDGES_PER_W + g * CHUNK
            esl = pl.ds(base, CHUNK)
            pltpu.sync_copy(col_hbm.at[esl], col_v)
            pltpu.sync_copy(row_hbm.at[esl], row_v)
            pltpu.sync_copy(w_hbm.at[esl], w_v)
            pltpu.async_copy(sup_hbm.at[col_v], rows_v, sem).wait()

            def scale_body(e, c2):
                wb = plsc.load_gather(w_v, [jnp.full((16,), e, jnp.int32)])
                for j in range(nvec):
                    fs = pl.ds(j * 16, 16)
                    rows_v[e, fs] = rows_v[e, fs] * wb
                return c2

            lax.fori_loop(0, CHUNK, scale_body, 0, unroll=2)
            pltpu.sync_copy(rows_v, acc_sh.at[row_v], add=True)
            return carry

        lax.fori_loop(0, CHUNKS_PER_W, chunk_body, 0)
        plsc.subcore_barrier()
        pltpu.sync_copy(acc_sh.at[slab], out_hbm.at[cid, slab])

    return pl.kernel(
        body,
        out_type=jax.ShapeDtypeStruct((NC, N_PAD, d), jnp.float32),
        mesh=mesh,
        compiler_params=pltpu.CompilerParams(
            needs_layout_passes=False, use_tc_tiling_on_sc=False),
        scratch_types=[
            pltpu.VMEM((CHUNK,), jnp.int32),
            pltpu.VMEM((CHUNK,), jnp.int32),
            pltpu.VMEM((CHUNK,), jnp.float32),
            pltpu.VMEM((CHUNK, d), jnp.float32),
            pltpu.VMEM_SHARED((N_PAD, d), jnp.float32),
            pltpu.SemaphoreType.DMA,
        ],
    )


_agg_d1 = _make_agg(D1)
_agg_d2 = _make_agg(D2)


@jax.jit
def _run(node_features, edge_index, edge_weight, W1, b1, W2, b2):
    row = edge_index[0].astype(jnp.int32)
    col = edge_index[1].astype(jnp.int32)
    x = jnp.pad(node_features, ((0, N_PAD - N), (0, 0)))
    rowp = jnp.pad(row, (0, E_PAD - E))
    colp = jnp.pad(col, (0, E_PAD - E))
    wp = jnp.pad(edge_weight.astype(jnp.float32), (0, E_PAD - E))
    w2p = jnp.pad(W2, ((0, D2 - C), (0, 0)))
    b2p = jnp.pad(b2, (0, D2 - C))
    zeros1 = jnp.zeros((N_PAD, D1), jnp.float32)
    zeros2 = jnp.zeros((N_PAD, D2), jnp.float32)

    sup1 = _linear(x, W1, b1)
    p1 = _agg_d1(sup1, colp, rowp, wp, zeros1)
    sup2 = _fused_linear(p1[0], p1[1], w2p, b2p)
    p2 = _agg_d2(sup2, colp, rowp, wp, zeros2)
    out = _logsoftmax(p2[0], p2[1])
    return out[:N]


def kernel(node_features, edge_index, edge_weight, W1, b1, W2, b2):
    return _run(node_features, edge_index, edge_weight, W1, b1, W2, b2)


# speedup vs baseline: 4.0504x; 1.4124x over previous
import functools

import jax
import jax.numpy as jnp
from jax import lax
from jax.experimental import pallas as pl
from jax.experimental.pallas import tpu as pltpu
from jax.experimental.pallas import tpu_sc as plsc

N = 10000
N_PAD = 10240
E = 320000
E_PAD = 327680
CHUNK = 128
D1 = 128
C = 40
D2 = 48
NC = 2
NS = 16
NW = NC * NS
EDGES_PER_W = E_PAD // NW
CHUNKS_PER_W = EDGES_PER_W // CHUNK
ROWS_PER_TILE = N_PAD // NS


def _linear_block(x_ref, w_ref, b_ref, o_ref):
    o_ref[...] = lax.dot_general(
        x_ref[...], w_ref[...], (((1,), (1,)), ((), ())),
        preferred_element_type=jnp.float32) + b_ref[...]


def _linear(x, w, b, bm=512):
    m, k = x.shape
    dout = w.shape[0]
    return pl.pallas_call(
        _linear_block,
        grid=(m // bm,),
        in_specs=[
            pl.BlockSpec((bm, k), lambda i: (i, 0)),
            pl.BlockSpec((dout, k), lambda i: (0, 0)),
            pl.BlockSpec((1, dout), lambda i: (0, 0)),
        ],
        out_specs=pl.BlockSpec((bm, dout), lambda i: (i, 0)),
        out_shape=jax.ShapeDtypeStruct((m, dout), jnp.float32),
    )(x, w, b.reshape(1, dout))


def _fused_linear_block(pa_ref, pb_ref, w_ref, b_ref, o_ref):
    x = jnp.maximum(pa_ref[...] + pb_ref[...], 0.0)
    o_ref[...] = lax.dot_general(
        x, w_ref[...], (((1,), (1,)), ((), ())),
        preferred_element_type=jnp.float32) + b_ref[...]


def _fused_linear(pa, pb, w, b, bm=512):
    m, k = pa.shape
    dout = w.shape[0]
    return pl.pallas_call(
        _fused_linear_block,
        grid=(m // bm,),
        in_specs=[
            pl.BlockSpec((bm, k), lambda i: (i, 0)),
            pl.BlockSpec((bm, k), lambda i: (i, 0)),
            pl.BlockSpec((dout, k), lambda i: (0, 0)),
            pl.BlockSpec((1, dout), lambda i: (0, 0)),
        ],
        out_specs=pl.BlockSpec((bm, dout), lambda i: (i, 0)),
        out_shape=jax.ShapeDtypeStruct((m, dout), jnp.float32),
    )(pa, pb, w, b.reshape(1, dout))


def _logsoftmax_block(pa_ref, pb_ref, o_ref):
    z = pa_ref[...] + pb_ref[...]
    colid = lax.broadcasted_iota(jnp.int32, z.shape, 1)
    valid = colid < C
    zm = jnp.where(valid, z, -jnp.inf)
    mx = jnp.max(zm, axis=1, keepdims=True)
    ex = jnp.where(valid, jnp.exp(z - mx), 0.0)
    s = jnp.sum(ex, axis=1, keepdims=True)
    o_ref[...] = (z - mx - jnp.log(s))[:, :C]


def _logsoftmax(pa, pb, bm=512):
    m, k = pa.shape
    return pl.pallas_call(
        _logsoftmax_block,
        grid=(m // bm,),
        in_specs=[
            pl.BlockSpec((bm, k), lambda i: (i, 0)),
            pl.BlockSpec((bm, k), lambda i: (i, 0)),
        ],
        out_specs=pl.BlockSpec((bm, C), lambda i: (i, 0)),
        out_shape=jax.ShapeDtypeStruct((m, C), jnp.float32),
    )(pa, pb)


def _make_agg(d):
    nvec = d // 16
    mesh = plsc.VectorSubcoreMesh(core_axis_name="c", subcore_axis_name="s")
    nch = CHUNKS_PER_W

    def body(sup_hbm, col_hbm, row_hbm, w_hbm, zero_hbm, out_hbm,
             col0, col1, row0, row1, w_all, rows0, rows1, acc_sh,
             sem_i0, sem_i1, sem_g0, sem_g1, sem_s0, sem_s1):
        cid = lax.axis_index("c")
        sid = lax.axis_index("s")
        wid = sid * NC + cid
        slab = pl.ds(sid * ROWS_PER_TILE, ROWS_PER_TILE)
        pltpu.sync_copy(zero_hbm.at[slab], acc_sh.at[slab])
        cbase = wid * nch
        pltpu.sync_copy(w_hbm.at[pl.ds(cbase, nch)], w_all)

        rows = (rows0, rows1)
        cols = (col0, col1)
        rws = (row0, row1)
        sem_i = (sem_i0, sem_i1)
        sem_g = (sem_g0, sem_g1)
        sem_s = (sem_s0, sem_s1)

        def issue_idx(g, b):
            sl = pl.ds(cbase + g, 1)
            pltpu.async_copy(col_hbm.at[sl], cols[b], sem_i[b])
            pltpu.async_copy(row_hbm.at[sl], rws[b], sem_i[b])

        def wait_idx(g, b):
            sl = pl.ds(cbase + g, 1)
            pltpu.make_async_copy(col_hbm.at[sl], cols[b], sem_i[b]).wait()
            pltpu.make_async_copy(row_hbm.at[sl], rws[b], sem_i[b]).wait()

        plsc.subcore_barrier()
        issue_idx(0, 0)
        wait_idx(0, 0)
        pltpu.async_copy(sup_hbm.at[cols[0].at[0]], rows0, sem_g0)
        issue_idx(1, 1)

        def scale_chunk(g, rbuf):
            def group_body(i0, c):
                w16 = w_all[g, pl.ds(i0 * 16, 16)]
                for i in range(16):
                    wb = lax.gather(
                        w16, jnp.full((16, 1), i, jnp.int32),
                        lax.GatherDimensionNumbers(
                            offset_dims=(), collapsed_slice_dims=(0,),
                            start_index_map=(0,)),
                        (1,), mode=lax.GatherScatterMode.PROMISE_IN_BOUNDS)
                    for j in range(nvec):
                        fs = pl.ds(j * 16, 16)
                        rbuf[i0 * 16 + i, fs] = rbuf[i0 * 16 + i, fs] * wb
                return c
            lax.fori_loop(0, CHUNK // 16, group_body, 0)

        def step(g, b):
            rbuf, rother = rows[b], rows[1 - b]
            pltpu.make_async_copy(sup_hbm.at[cols[b].at[0]], rbuf,
                                  sem_g[b]).wait()
            scale_chunk(g, rbuf)
            pltpu.async_copy(rbuf, acc_sh.at[rws[b].at[0]], sem_s[b], add=True)

            @pl.when(g + 1 < nch)
            def _():
                wait_idx(g + 1, 1 - b)

                @pl.when(g >= 1)
                def _():
                    pltpu.make_async_copy(rother, acc_sh.at[rws[1 - b].at[0]],
                                          sem_s[1 - b]).wait()

                pltpu.async_copy(sup_hbm.at[cols[1 - b].at[0]], rother,
                                 sem_g[1 - b])

            @pl.when(g + 2 < nch)
            def _():
                issue_idx(g + 2, b)

        def outer(g2, c):
            step(g2 * 2, 0)
            step(g2 * 2 + 1, 1)
            return c

        lax.fori_loop(0, nch // 2, outer, 0)
        pltpu.make_async_copy(rows0, acc_sh.at[rws[0].at[0]], sem_s0).wait()
        pltpu.make_async_copy(rows1, acc_sh.at[rws[1].at[0]], sem_s1).wait()
        plsc.subcore_barrier()
        pltpu.sync_copy(acc_sh.at[slab], out_hbm.at[cid, slab])

    return pl.kernel(
        body,
        out_type=jax.ShapeDtypeStruct((NC, N_PAD, d), jnp.float32),
        mesh=mesh,
        compiler_params=pltpu.CompilerParams(
            needs_layout_passes=False, use_tc_tiling_on_sc=False),
        scratch_types=[
            pltpu.VMEM((1, CHUNK), jnp.int32),
            pltpu.VMEM((1, CHUNK), jnp.int32),
            pltpu.VMEM((1, CHUNK), jnp.int32),
            pltpu.VMEM((1, CHUNK), jnp.int32),
            pltpu.VMEM((CHUNKS_PER_W, CHUNK), jnp.float32),
            pltpu.VMEM((CHUNK, d), jnp.float32),
            pltpu.VMEM((CHUNK, d), jnp.float32),
            pltpu.VMEM_SHARED((N_PAD, d), jnp.float32),
            pltpu.SemaphoreType.DMA,
            pltpu.SemaphoreType.DMA,
            pltpu.SemaphoreType.DMA,
            pltpu.SemaphoreType.DMA,
            pltpu.SemaphoreType.DMA,
            pltpu.SemaphoreType.DMA,
        ],
    )


_agg_d1 = _make_agg(D1)
_agg_d2 = _make_agg(D2)


@jax.jit
def _run(node_features, edge_index, edge_weight, W1, b1, W2, b2):
    row = edge_index[0].astype(jnp.int32)
    col = edge_index[1].astype(jnp.int32)
    x = jnp.pad(node_features, ((0, N_PAD - N), (0, 0)))
    rowp = jnp.pad(row, (0, E_PAD - E)).reshape(NW * CHUNKS_PER_W, CHUNK)
    colp = jnp.pad(col, (0, E_PAD - E)).reshape(NW * CHUNKS_PER_W, CHUNK)
    wp = jnp.pad(edge_weight.astype(jnp.float32),
                 (0, E_PAD - E)).reshape(NW * CHUNKS_PER_W, CHUNK)
    w2p = jnp.pad(W2, ((0, D2 - C), (0, 0)))
    b2p = jnp.pad(b2, (0, D2 - C))
    zeros1 = jnp.zeros((N_PAD, D1), jnp.float32)
    zeros2 = jnp.zeros((N_PAD, D2), jnp.float32)

    sup1 = _linear(x, W1, b1)
    p1 = _agg_d1(sup1, colp, rowp, wp, zeros1)
    sup2 = _fused_linear(p1[0], p1[1], w2p, b2p)
    p2 = _agg_d2(sup2, colp, rowp, wp, zeros2)
    out = _logsoftmax(p2[0], p2[1])
    return out[:N]


def kernel(node_features, edge_index, edge_weight, W1, b1, W2, b2):
    return _run(node_features, edge_index, edge_weight, W1, b1, W2, b2)


# speedup vs baseline: 5.2556x; 1.2975x over previous
import functools

import jax
import jax.numpy as jnp
from jax import lax
from jax.experimental import pallas as pl
from jax.experimental.pallas import tpu as pltpu
from jax.experimental.pallas import tpu_sc as plsc

N = 10000
N_PAD = 10240
E = 320000
E_PAD = 327680
CHUNK = 128
D1 = 128
DH = 64
C = 40
D2 = 48
NC = 2
NS = 16
NW = NC * NS
NCHUNKS = E_PAD // CHUNK
ROWS_PER_TILE = N_PAD // NS


def _linear_split_block(x_ref, w_ref, b_ref, o_ref):
    res = lax.dot_general(
        x_ref[...], w_ref[...], (((1,), (1,)), ((), ())),
        preferred_element_type=jnp.float32) + b_ref[...]
    o_ref[0] = res[:, :DH]
    o_ref[1] = res[:, DH:]


def _linear_split(x, w, b, bm=512):
    m, k = x.shape
    dout = w.shape[0]
    return pl.pallas_call(
        _linear_split_block,
        grid=(m // bm,),
        in_specs=[
            pl.BlockSpec((bm, k), lambda i: (i, 0)),
            pl.BlockSpec((dout, k), lambda i: (0, 0)),
            pl.BlockSpec((1, dout), lambda i: (0, 0)),
        ],
        out_specs=pl.BlockSpec((NC, bm, DH), lambda i: (0, i, 0)),
        out_shape=jax.ShapeDtypeStruct((NC, m, DH), jnp.float32),
    )(x, w, b.reshape(1, dout))


def _fused_linear_block(pa_ref, pb_ref, wa_ref, wb_ref, b_ref, o_ref):
    ha = jnp.maximum(pa_ref[...], 0.0)
    hb = jnp.maximum(pb_ref[...], 0.0)
    dn = (((1,), (1,)), ((), ()))
    o_ref[...] = (
        lax.dot_general(ha, wa_ref[...], dn, preferred_element_type=jnp.float32)
        + lax.dot_general(hb, wb_ref[...], dn, preferred_element_type=jnp.float32)
        + b_ref[...])


def _fused_linear(pa, pb, wa, wb, b, bm=512):
    m, k = pa.shape
    dout = wa.shape[0]
    return pl.pallas_call(
        _fused_linear_block,
        grid=(m // bm,),
        in_specs=[
            pl.BlockSpec((bm, k), lambda i: (i, 0)),
            pl.BlockSpec((bm, k), lambda i: (i, 0)),
            pl.BlockSpec((dout, k), lambda i: (0, 0)),
            pl.BlockSpec((dout, k), lambda i: (0, 0)),
            pl.BlockSpec((1, dout), lambda i: (0, 0)),
        ],
        out_specs=pl.BlockSpec((bm, dout), lambda i: (i, 0)),
        out_shape=jax.ShapeDtypeStruct((m, dout), jnp.float32),
    )(pa, pb, wa, wb, b.reshape(1, dout))


def _logsoftmax_block(pa_ref, pb_ref, o_ref):
    z = pa_ref[...] + pb_ref[...]
    colid = lax.broadcasted_iota(jnp.int32, z.shape, 1)
    valid = colid < C
    zm = jnp.where(valid, z, -jnp.inf)
    mx = jnp.max(zm, axis=1, keepdims=True)
    ex = jnp.where(valid, jnp.exp(z - mx), 0.0)
    s = jnp.sum(ex, axis=1, keepdims=True)
    o_ref[...] = (z - mx - jnp.log(s))[:, :C]


def _logsoftmax(pa, pb, bm=512):
    m, k = pa.shape
    return pl.pallas_call(
        _logsoftmax_block,
        grid=(m // bm,),
        in_specs=[
            pl.BlockSpec((bm, k), lambda i: (i, 0)),
            pl.BlockSpec((bm, k), lambda i: (i, 0)),
        ],
        out_specs=pl.BlockSpec((bm, C), lambda i: (i, 0)),
        out_shape=jax.ShapeDtypeStruct((m, C), jnp.float32),
    )(pa, pb)


def _make_agg(d, feature_split):
    nvec = d // 16
    mesh = plsc.VectorSubcoreMesh(core_axis_name="c", subcore_axis_name="s")
    nch = NCHUNKS // NS if feature_split else NCHUNKS // NW

    def body(sup_hbm, col_hbm, row_hbm, w_hbm, zero_hbm, out_hbm,
             col0, col1, row0, row1, w_all, rows0, rows1, sup_sh, acc_sh,
             sem_i0, sem_i1, sem_g0, sem_g1, sem_s0, sem_s1):
        cid = lax.axis_index("c")
        sid = lax.axis_index("s")
        slab = pl.ds(sid * ROWS_PER_TILE, ROWS_PER_TILE)
        pltpu.sync_copy(zero_hbm.at[slab], acc_sh.at[slab])
        if feature_split:
            pltpu.sync_copy(sup_hbm.at[cid, slab], sup_sh.at[slab])
            cbase = sid * nch
        else:
            pltpu.sync_copy(sup_hbm.at[slab], sup_sh.at[slab])
            cbase = (sid * NC + cid) * nch
        pltpu.sync_copy(w_hbm.at[pl.ds(cbase, nch)], w_all)

        rows = (rows0, rows1)
        cols = (col0, col1)
        rws = (row0, row1)
        sem_i = (sem_i0, sem_i1)
        sem_g = (sem_g0, sem_g1)
        sem_s = (sem_s0, sem_s1)

        def issue_idx(g, b):
            sl = pl.ds(cbase + g, 1)
            pltpu.async_copy(col_hbm.at[sl], cols[b], sem_i[b])
            pltpu.async_copy(row_hbm.at[sl], rws[b], sem_i[b])

        def wait_idx(g, b):
            sl = pl.ds(cbase + g, 1)
            pltpu.make_async_copy(col_hbm.at[sl], cols[b], sem_i[b]).wait()
            pltpu.make_async_copy(row_hbm.at[sl], rws[b], sem_i[b]).wait()

        plsc.subcore_barrier()
        issue_idx(0, 0)
        wait_idx(0, 0)
        pltpu.async_copy(sup_sh.at[cols[0].at[0]], rows0, sem_g0)
        issue_idx(1, 1)

        def scale_chunk(g, rbuf):
            def group_body(i0, c):
                w16 = w_all[g, pl.ds(i0 * 16, 16)]
                for i in range(16):
                    wb = lax.gather(
                        w16, jnp.full((16, 1), i, jnp.int32),
                        lax.GatherDimensionNumbers(
                            offset_dims=(), collapsed_slice_dims=(0,),
                            start_index_map=(0,)),
                        (1,), mode=lax.GatherScatterMode.PROMISE_IN_BOUNDS)
                    for j in range(nvec):
                        fs = pl.ds(j * 16, 16)
                        rbuf[i0 * 16 + i, fs] = rbuf[i0 * 16 + i, fs] * wb
                return c
            lax.fori_loop(0, CHUNK // 16, group_body, 0)

        def step(g, b):
            rbuf, rother = rows[b], rows[1 - b]
            pltpu.make_async_copy(sup_sh.at[cols[b].at[0]], rbuf,
                                  sem_g[b]).wait()
            scale_chunk(g, rbuf)
            pltpu.async_copy(rbuf, acc_sh.at[rws[b].at[0]], sem_s[b], add=True)

            @pl.when(g + 1 < nch)
            def _():
                wait_idx(g + 1, 1 - b)

                @pl.when(g >= 1)
                def _():
                    pltpu.make_async_copy(rother, acc_sh.at[rws[1 - b].at[0]],
                                          sem_s[1 - b]).wait()

                pltpu.async_copy(sup_sh.at[cols[1 - b].at[0]], rother,
                                 sem_g[1 - b])

            @pl.when(g + 2 < nch)
            def _():
                issue_idx(g + 2, b)

        def outer(g2, c):
            step(g2 * 2, 0)
            step(g2 * 2 + 1, 1)
            return c

        lax.fori_loop(0, nch // 2, outer, 0)
        pltpu.make_async_copy(rows0, acc_sh.at[rws[0].at[0]], sem_s0).wait()
        pltpu.make_async_copy(rows1, acc_sh.at[rws[1].at[0]], sem_s1).wait()
        plsc.subcore_barrier()
        pltpu.sync_copy(acc_sh.at[slab], out_hbm.at[cid, slab])

    return pl.kernel(
        body,
        out_type=jax.ShapeDtypeStruct((NC, N_PAD, d), jnp.float32),
        mesh=mesh,
        compiler_params=pltpu.CompilerParams(
            needs_layout_passes=False, use_tc_tiling_on_sc=False),
        scratch_types=[
            pltpu.VMEM((1, CHUNK), jnp.int32),
            pltpu.VMEM((1, CHUNK), jnp.int32),
            pltpu.VMEM((1, CHUNK), jnp.int32),
            pltpu.VMEM((1, CHUNK), jnp.int32),
            pltpu.VMEM((nch, CHUNK), jnp.float32),
            pltpu.VMEM((CHUNK, d), jnp.float32),
            pltpu.VMEM((CHUNK, d), jnp.float32),
            pltpu.VMEM_SHARED((N_PAD, d), jnp.float32),
            pltpu.VMEM_SHARED((N_PAD, d), jnp.float32),
            pltpu.SemaphoreType.DMA,
            pltpu.SemaphoreType.DMA,
            pltpu.SemaphoreType.DMA,
            pltpu.SemaphoreType.DMA,
            pltpu.SemaphoreType.DMA,
            pltpu.SemaphoreType.DMA,
        ],
    )


_agg_l1 = _make_agg(DH, feature_split=True)
_agg_l2 = _make_agg(D2, feature_split=False)


@jax.jit
def _run(node_features, edge_index, edge_weight, W1, b1, W2, b2):
    row = edge_index[0].astype(jnp.int32)
    col = edge_index[1].astype(jnp.int32)
    x = jnp.pad(node_features, ((0, N_PAD - N), (0, 0)))
    rowp = jnp.pad(row, (0, E_PAD - E)).reshape(NCHUNKS, CHUNK)
    colp = jnp.pad(col, (0, E_PAD - E)).reshape(NCHUNKS, CHUNK)
    wp = jnp.pad(edge_weight.astype(jnp.float32),
                 (0, E_PAD - E)).reshape(NCHUNKS, CHUNK)
    w2p = jnp.pad(W2, ((0, D2 - C), (0, 0)))
    b2p = jnp.pad(b2, (0, D2 - C))
    w2a = w2p[:, :DH]
    w2b = w2p[:, DH:]
    zeros1 = jnp.zeros((N_PAD, DH), jnp.float32)
    zeros2 = jnp.zeros((N_PAD, D2), jnp.float32)

    sup1 = _linear_split(x, W1, b1)
    p1 = _agg_l1(sup1, colp, rowp, wp, zeros1)
    sup2 = _fused_linear(p1[0], p1[1], w2a, w2b, b2p)
    p2 = _agg_l2(sup2, colp, rowp, wp, zeros2)
    out = _logsoftmax(p2[0], p2[1])
    return out[:N]


def kernel(node_features, edge_index, edge_weight, W1, b1, W2, b2):
    return _run(node_features, edge_index, edge_weight, W1, b1, W2, b2)


# speedup vs baseline: 8.4414x; 1.6062x over previous
import functools

import jax
import jax.numpy as jnp
from jax import lax
from jax.experimental import pallas as pl
from jax.experimental.pallas import tpu as pltpu
from jax.experimental.pallas import tpu_sc as plsc

N = 10000
N_PAD = 10240
E = 320000
E_PAD = 327680
CHUNK = 128
D1 = 128
DH = 64
C = 40
D2 = 48
NC = 2
NS = 16
NW = NC * NS
NCHUNKS = E_PAD // CHUNK
ROWS_PER_TILE = N_PAD // NS


def _linear_split_block(x_ref, w_ref, b_ref, o_ref):
    res = lax.dot_general(
        x_ref[...], w_ref[...], (((1,), (1,)), ((), ())),
        preferred_element_type=jnp.float32) + b_ref[...]
    o_ref[0] = res[:, :DH]
    o_ref[1] = res[:, DH:]


def _linear_split(x, w, b, bm=512):
    m, k = x.shape
    dout = w.shape[0]
    return pl.pallas_call(
        _linear_split_block,
        grid=(m // bm,),
        in_specs=[
            pl.BlockSpec((bm, k), lambda i: (i, 0)),
            pl.BlockSpec((dout, k), lambda i: (0, 0)),
            pl.BlockSpec((1, dout), lambda i: (0, 0)),
        ],
        out_specs=pl.BlockSpec((NC, bm, DH), lambda i: (0, i, 0)),
        out_shape=jax.ShapeDtypeStruct((NC, m, DH), jnp.float32),
    )(x, w, b.reshape(1, dout))


def _fused_linear_block(pa_ref, pb_ref, wa_ref, wb_ref, b_ref, o_ref):
    ha = jnp.maximum(pa_ref[...], 0.0)
    hb = jnp.maximum(pb_ref[...], 0.0)
    dn = (((1,), (1,)), ((), ()))
    o_ref[...] = (
        lax.dot_general(ha, wa_ref[...], dn, preferred_element_type=jnp.float32)
        + lax.dot_general(hb, wb_ref[...], dn, preferred_element_type=jnp.float32)
        + b_ref[...])


def _fused_linear(pa, pb, wa, wb, b, bm=512):
    m, k = pa.shape
    dout = wa.shape[0]
    return pl.pallas_call(
        _fused_linear_block,
        grid=(m // bm,),
        in_specs=[
            pl.BlockSpec((bm, k), lambda i: (i, 0)),
            pl.BlockSpec((bm, k), lambda i: (i, 0)),
            pl.BlockSpec((dout, k), lambda i: (0, 0)),
            pl.BlockSpec((dout, k), lambda i: (0, 0)),
            pl.BlockSpec((1, dout), lambda i: (0, 0)),
        ],
        out_specs=pl.BlockSpec((bm, dout), lambda i: (i, 0)),
        out_shape=jax.ShapeDtypeStruct((m, dout), jnp.float32),
    )(pa, pb, wa, wb, b.reshape(1, dout))


def _logsoftmax_block(pa_ref, pb_ref, o_ref):
    z = pa_ref[...] + pb_ref[...]
    colid = lax.broadcasted_iota(jnp.int32, z.shape, 1)
    valid = colid < C
    zm = jnp.where(valid, z, -jnp.inf)
    mx = jnp.max(zm, axis=1, keepdims=True)
    ex = jnp.where(valid, jnp.exp(z - mx), 0.0)
    s = jnp.sum(ex, axis=1, keepdims=True)
    o_ref[...] = (z - mx - jnp.log(s))[:, :C]


def _logsoftmax(pa, pb, bm=512):
    m, k = pa.shape
    return pl.pallas_call(
        _logsoftmax_block,
        grid=(m // bm,),
        in_specs=[
            pl.BlockSpec((bm, k), lambda i: (i, 0)),
            pl.BlockSpec((bm, k), lambda i: (i, 0)),
        ],
        out_specs=pl.BlockSpec((bm, C), lambda i: (i, 0)),
        out_shape=jax.ShapeDtypeStruct((m, C), jnp.float32),
    )(pa, pb)


def _make_agg(d, feature_split):
    nvec = d // 16
    mesh = plsc.VectorSubcoreMesh(core_axis_name="c", subcore_axis_name="s")
    nch = NCHUNKS // NS if feature_split else NCHUNKS // NW

    def body(sup_hbm, col_hbm, row_hbm, w_hbm, zero_hbm, out_hbm,
             col0, col1, row0, row1, w_all, rows0, rows1, sup_sh, acc_sh,
             sem_i0, sem_i1, sem_g0, sem_g1, sem_s0, sem_s1):
        cid = lax.axis_index("c")
        sid = lax.axis_index("s")
        slab = pl.ds(sid * ROWS_PER_TILE, ROWS_PER_TILE)
        pltpu.sync_copy(zero_hbm.at[slab], acc_sh.at[slab])
        if feature_split:
            pltpu.sync_copy(sup_hbm.at[cid, slab], sup_sh.at[slab])
            cbase = sid * nch
        else:
            pltpu.sync_copy(sup_hbm.at[slab], sup_sh.at[slab])
            cbase = (sid * NC + cid) * nch
        pltpu.sync_copy(w_hbm.at[pl.ds(cbase, nch)], w_all)

        rows = (rows0, rows1)
        cols = (col0, col1)
        rws = (row0, row1)
        sem_i = (sem_i0, sem_i1)
        sem_g = (sem_g0, sem_g1)
        sem_s = (sem_s0, sem_s1)

        def issue_idx(g, b):
            sl = pl.ds(cbase + g, 1)
            pltpu.async_copy(col_hbm.at[sl], cols[b], sem_i[b])
            pltpu.async_copy(row_hbm.at[sl], rws[b], sem_i[b])

        def wait_idx(g, b):
            sl = pl.ds(cbase + g, 1)
            pltpu.make_async_copy(col_hbm.at[sl], cols[b], sem_i[b]).wait()
            pltpu.make_async_copy(row_hbm.at[sl], rws[b], sem_i[b]).wait()

        plsc.subcore_barrier()
        issue_idx(0, 0)
        wait_idx(0, 0)
        pltpu.async_copy(sup_sh.at[cols[0].at[0]], rows0, sem_g0)
        issue_idx(1, 1)

        def scale_chunk(g, rbuf):
            for i0 in range(CHUNK // 16):
                w16 = w_all[g, pl.ds(i0 * 16, 16)]
                for i in range(16):
                    wb = lax.gather(
                        w16, jnp.full((16, 1), i, jnp.int32),
                        lax.GatherDimensionNumbers(
                            offset_dims=(), collapsed_slice_dims=(0,),
                            start_index_map=(0,)),
                        (1,), mode=lax.GatherScatterMode.PROMISE_IN_BOUNDS)
                    e = i0 * 16 + i
                    for j in range(nvec):
                        fs = pl.ds(j * 16, 16)
                        rbuf[e, fs] = rbuf[e, fs] * wb

        def step(g, b):
            rbuf, rother = rows[b], rows[1 - b]
            pltpu.make_async_copy(sup_sh.at[cols[b].at[0]], rbuf,
                                  sem_g[b]).wait()
            scale_chunk(g, rbuf)
            pltpu.async_copy(rbuf, acc_sh.at[rws[b].at[0]], sem_s[b], add=True)

            @pl.when(g + 1 < nch)
            def _():
                wait_idx(g + 1, 1 - b)

                @pl.when(g >= 1)
                def _():
                    pltpu.make_async_copy(rother, acc_sh.at[rws[1 - b].at[0]],
                                          sem_s[1 - b]).wait()

                pltpu.async_copy(sup_sh.at[cols[1 - b].at[0]], rother,
                                 sem_g[1 - b])

            @pl.when(g + 2 < nch)
            def _():
                issue_idx(g + 2, b)

        def outer(g2, c):
            step(g2 * 2, 0)
            step(g2 * 2 + 1, 1)
            return c

        lax.fori_loop(0, nch // 2, outer, 0)
        pltpu.make_async_copy(rows0, acc_sh.at[rws[0].at[0]], sem_s0).wait()
        pltpu.make_async_copy(rows1, acc_sh.at[rws[1].at[0]], sem_s1).wait()
        plsc.subcore_barrier()
        pltpu.sync_copy(acc_sh.at[slab], out_hbm.at[cid, slab])

    return pl.kernel(
        body,
        out_type=jax.ShapeDtypeStruct((NC, N_PAD, d), jnp.float32),
        mesh=mesh,
        compiler_params=pltpu.CompilerParams(
            needs_layout_passes=False, use_tc_tiling_on_sc=False),
        scratch_types=[
            pltpu.VMEM((1, CHUNK), jnp.int32),
            pltpu.VMEM((1, CHUNK), jnp.int32),
            pltpu.VMEM((1, CHUNK), jnp.int32),
            pltpu.VMEM((1, CHUNK), jnp.int32),
            pltpu.VMEM((nch, CHUNK), jnp.float32),
            pltpu.VMEM((CHUNK, d), jnp.float32),
            pltpu.VMEM((CHUNK, d), jnp.float32),
            pltpu.VMEM_SHARED((N_PAD, d), jnp.float32),
            pltpu.VMEM_SHARED((N_PAD, d), jnp.float32),
            pltpu.SemaphoreType.DMA,
            pltpu.SemaphoreType.DMA,
            pltpu.SemaphoreType.DMA,
            pltpu.SemaphoreType.DMA,
            pltpu.SemaphoreType.DMA,
            pltpu.SemaphoreType.DMA,
        ],
    )


_agg_l1 = _make_agg(DH, feature_split=True)
_agg_l2 = _make_agg(D2, feature_split=False)


@jax.jit
def _run(node_features, edge_index, edge_weight, W1, b1, W2, b2):
    row = edge_index[0].astype(jnp.int32)
    col = edge_index[1].astype(jnp.int32)
    x = jnp.pad(node_features, ((0, N_PAD - N), (0, 0)))
    rowp = jnp.pad(row, (0, E_PAD - E)).reshape(NCHUNKS, CHUNK)
    colp = jnp.pad(col, (0, E_PAD - E)).reshape(NCHUNKS, CHUNK)
    wp = jnp.pad(edge_weight.astype(jnp.float32),
                 (0, E_PAD - E)).reshape(NCHUNKS, CHUNK)
    w2p = jnp.pad(W2, ((0, D2 - C), (0, 0)))
    b2p = jnp.pad(b2, (0, D2 - C))
    w2a = w2p[:, :DH]
    w2b = w2p[:, DH:]
    zeros1 = jnp.zeros((N_PAD, DH), jnp.float32)
    zeros2 = jnp.zeros((N_PAD, D2), jnp.float32)

    sup1 = _linear_split(x, W1, b1)
    p1 = _agg_l1(sup1, colp, rowp, wp, zeros1)
    sup2 = _fused_linear(p1[0], p1[1], w2a, w2b, b2p)
    p2 = _agg_l2(sup2, colp, rowp, wp, zeros2)
    out = _logsoftmax(p2[0], p2[1])
    return out[:N]


def kernel(node_features, edge_index, edge_weight, W1, b1, W2, b2):
    return _run(node_features, edge_index, edge_weight, W1, b1, W2, b2)


# speedup vs baseline: 8.9985x; 1.0660x over previous
import functools

import jax
import jax.numpy as jnp
from jax import lax
from jax.experimental import pallas as pl
from jax.experimental.pallas import tpu as pltpu
from jax.experimental.pallas import tpu_sc as plsc

N = 10000
N_PAD = 10240
E = 320000
E_PAD = 327680
CHUNK = 128
D1 = 128
DH = 64
C = 40
D2 = 48
NC = 2
NS = 16
NW = NC * NS
NCHUNKS = E_PAD // CHUNK
ROWS_PER_TILE = N_PAD // NS


def _linear_split_block(x_ref, w_ref, b_ref, o_ref):
    res = lax.dot_general(
        x_ref[...], w_ref[...], (((1,), (1,)), ((), ())),
        preferred_element_type=jnp.float32) + b_ref[...]
    o_ref[0] = res[:, :DH]
    o_ref[1] = res[:, DH:]


def _linear_split(x, w, b, bm=512):
    m, k = x.shape
    dout = w.shape[0]
    return pl.pallas_call(
        _linear_split_block,
        grid=(m // bm,),
        in_specs=[
            pl.BlockSpec((bm, k), lambda i: (i, 0)),
            pl.BlockSpec((dout, k), lambda i: (0, 0)),
            pl.BlockSpec((1, dout), lambda i: (0, 0)),
        ],
        out_specs=pl.BlockSpec((NC, bm, DH), lambda i: (0, i, 0)),
        out_shape=jax.ShapeDtypeStruct((NC, m, DH), jnp.float32),
    )(x, w, b.reshape(1, dout))


def _fused_linear_block(pa_ref, pb_ref, wa_ref, wb_ref, b_ref, o_ref):
    ha = jnp.maximum(pa_ref[0], 0.0)
    hb = jnp.maximum(pb_ref[0], 0.0)
    dn = (((1,), (1,)), ((), ()))
    o_ref[...] = (
        lax.dot_general(ha, wa_ref[...], dn, preferred_element_type=jnp.float32)
        + lax.dot_general(hb, wb_ref[...], dn, preferred_element_type=jnp.float32)
        + b_ref[...])


def _fused_linear(p, wa, wb, b, bm=512):
    _, m, k = p.shape
    dout = wa.shape[0]
    return pl.pallas_call(
        _fused_linear_block,
        grid=(m // bm,),
        in_specs=[
            pl.BlockSpec((1, bm, k), lambda i: (0, i, 0)),
            pl.BlockSpec((1, bm, k), lambda i: (1, i, 0)),
            pl.BlockSpec((dout, k), lambda i: (0, 0)),
            pl.BlockSpec((dout, k), lambda i: (0, 0)),
            pl.BlockSpec((1, dout), lambda i: (0, 0)),
        ],
        out_specs=pl.BlockSpec((bm, dout), lambda i: (i, 0)),
        out_shape=jax.ShapeDtypeStruct((m, dout), jnp.float32),
    )(p, p, wa, wb, b.reshape(1, dout))


def _logsoftmax_block(pa_ref, pb_ref, o_ref):
    z = pa_ref[0] + pb_ref[0]
    colid = lax.broadcasted_iota(jnp.int32, z.shape, 1)
    valid = colid < C
    zm = jnp.where(valid, z, -jnp.inf)
    mx = jnp.max(zm, axis=1, keepdims=True)
    ex = jnp.where(valid, jnp.exp(z - mx), 0.0)
    s = jnp.sum(ex, axis=1, keepdims=True)
    o_ref[...] = (z - mx - jnp.log(s))[:, :C]


def _logsoftmax(p, bm=400):
    k = p.shape[2]
    return pl.pallas_call(
        _logsoftmax_block,
        grid=(N // bm,),
        in_specs=[
            pl.BlockSpec((1, bm, k), lambda i: (0, i, 0)),
            pl.BlockSpec((1, bm, k), lambda i: (1, i, 0)),
        ],
        out_specs=pl.BlockSpec((bm, C), lambda i: (i, 0)),
        out_shape=jax.ShapeDtypeStruct((N, C), jnp.float32),
    )(p, p)


def _make_agg(d, feature_split):
    nvec = d // 16
    mesh = plsc.VectorSubcoreMesh(core_axis_name="c", subcore_axis_name="s")
    nch = NCHUNKS // NS if feature_split else NCHUNKS // NW

    def body(sup_hbm, ei_hbm, w_hbm, zero_hbm, out_hbm,
             col0, col1, row0, row1, w_all, rows0, rows1, sup_sh, acc_sh,
             sem_i0, sem_i1, sem_g0, sem_g1, sem_s0, sem_s1):
        cid = lax.axis_index("c")
        sid = lax.axis_index("s")
        slab = pl.ds(sid * ROWS_PER_TILE, ROWS_PER_TILE)
        pltpu.sync_copy(zero_hbm.at[slab], acc_sh.at[slab])
        if feature_split:
            pltpu.sync_copy(sup_hbm.at[cid, slab], sup_sh.at[slab])
            cbase = sid * nch
        else:
            pltpu.sync_copy(sup_hbm.at[slab], sup_sh.at[slab])
            cbase = (sid * NC + cid) * nch
        pltpu.sync_copy(w_hbm.at[pl.ds(cbase * CHUNK, nch * CHUNK)], w_all)

        rows = (rows0, rows1)
        cols = (col0, col1)
        rws = (row0, row1)
        sem_i = (sem_i0, sem_i1)
        sem_g = (sem_g0, sem_g1)
        sem_s = (sem_s0, sem_s1)

        def issue_idx(g, b):
            sl = pl.ds((cbase + g) * CHUNK, CHUNK)
            pltpu.async_copy(ei_hbm.at[pl.ds(1, 1), sl], cols[b], sem_i[b])
            pltpu.async_copy(ei_hbm.at[pl.ds(0, 1), sl], rws[b], sem_i[b])

        def wait_idx(g, b):
            sl = pl.ds((cbase + g) * CHUNK, CHUNK)
            pltpu.make_async_copy(ei_hbm.at[pl.ds(1, 1), sl], cols[b],
                                  sem_i[b]).wait()
            pltpu.make_async_copy(ei_hbm.at[pl.ds(0, 1), sl], rws[b],
                                  sem_i[b]).wait()

        plsc.subcore_barrier()
        issue_idx(0, 0)
        wait_idx(0, 0)
        pltpu.async_copy(sup_sh.at[cols[0].at[0]], rows0, sem_g0)
        issue_idx(1, 1)

        def scale_chunk(g, rbuf):
            wbase = g * CHUNK
            for i0 in range(CHUNK // 16):
                w16 = w_all[pl.ds(wbase + i0 * 16, 16)]
                for i in range(16):
                    wb = lax.gather(
                        w16, jnp.full((16, 1), i, jnp.int32),
                        lax.GatherDimensionNumbers(
                            offset_dims=(), collapsed_slice_dims=(0,),
                            start_index_map=(0,)),
                        (1,), mode=lax.GatherScatterMode.PROMISE_IN_BOUNDS)
                    e = i0 * 16 + i
                    for j in range(nvec):
                        fs = pl.ds(j * 16, 16)
                        rbuf[e, fs] = rbuf[e, fs] * wb

        def step(g, b):
            rbuf, rother = rows[b], rows[1 - b]
            pltpu.make_async_copy(sup_sh.at[cols[b].at[0]], rbuf,
                                  sem_g[b]).wait()
            scale_chunk(g, rbuf)
            pltpu.async_copy(rbuf, acc_sh.at[rws[b].at[0]], sem_s[b], add=True)

            @pl.when(g + 1 < nch)
            def _():
                wait_idx(g + 1, 1 - b)

                @pl.when(g >= 1)
                def _():
                    pltpu.make_async_copy(rother, acc_sh.at[rws[1 - b].at[0]],
                                          sem_s[1 - b]).wait()

                pltpu.async_copy(sup_sh.at[cols[1 - b].at[0]], rother,
                                 sem_g[1 - b])

            @pl.when(g + 2 < nch)
            def _():
                issue_idx(g + 2, b)

        def outer(g2, c):
            step(g2 * 2, 0)
            step(g2 * 2 + 1, 1)
            return c

        lax.fori_loop(0, nch // 2, outer, 0)
        pltpu.make_async_copy(rows0, acc_sh.at[rws[0].at[0]], sem_s0).wait()
        pltpu.make_async_copy(rows1, acc_sh.at[rws[1].at[0]], sem_s1).wait()
        plsc.subcore_barrier()
        pltpu.sync_copy(acc_sh.at[slab], out_hbm.at[cid, slab])

    return pl.kernel(
        body,
        out_type=jax.ShapeDtypeStruct((NC, N_PAD, d), jnp.float32),
        mesh=mesh,
        compiler_params=pltpu.CompilerParams(
            needs_layout_passes=False, use_tc_tiling_on_sc=False),
        scratch_types=[
            pltpu.VMEM((1, CHUNK), jnp.int32),
            pltpu.VMEM((1, CHUNK), jnp.int32),
            pltpu.VMEM((1, CHUNK), jnp.int32),
            pltpu.VMEM((1, CHUNK), jnp.int32),
            pltpu.VMEM((nch * CHUNK,), jnp.float32),
            pltpu.VMEM((CHUNK, d), jnp.float32),
            pltpu.VMEM((CHUNK, d), jnp.float32),
            pltpu.VMEM_SHARED((N_PAD, d), jnp.float32),
            pltpu.VMEM_SHARED((N_PAD, d), jnp.float32),
            pltpu.SemaphoreType.DMA,
            pltpu.SemaphoreType.DMA,
            pltpu.SemaphoreType.DMA,
            pltpu.SemaphoreType.DMA,
            pltpu.SemaphoreType.DMA,
            pltpu.SemaphoreType.DMA,
        ],
    )


_agg_l1 = _make_agg(DH, feature_split=True)
_agg_l2 = _make_agg(D2, feature_split=False)


@jax.jit
def _run(node_features, edge_index, edge_weight, W1, b1, W2, b2):
    ei = jnp.pad(edge_index.astype(jnp.int32), ((0, 0), (0, E_PAD - E)))
    x = jnp.pad(node_features, ((0, N_PAD - N), (0, 0)))
    wp = jnp.pad(edge_weight.astype(jnp.float32), (0, E_PAD - E))
    w2p = jnp.pad(W2, ((0, D2 - C), (0, 0)))
    b2p = jnp.pad(b2, (0, D2 - C))
    w2a = w2p[:, :DH]
    w2b = w2p[:, DH:]
    zeros1 = jnp.zeros((N_PAD, DH), jnp.float32)
    zeros2 = jnp.zeros((N_PAD, D2), jnp.float32)

    sup1 = _linear_split(x, W1, b1)
    p1 = _agg_l1(sup1, ei, wp, zeros1)
    sup2 = _fused_linear(p1, w2a, w2b, b2p)
    p2 = _agg_l2(sup2, ei, wp, zeros2)
    return _logsoftmax(p2)


def kernel(node_features, edge_index, edge_weight, W1, b1, W2, b2):
    return _run(node_features, edge_index, edge_weight, W1, b1, W2, b2)


# speedup vs baseline: 9.9295x; 1.1035x over previous
import functools

import jax
import jax.numpy as jnp
from jax import lax
from jax.experimental import pallas as pl
from jax.experimental.pallas import tpu as pltpu
from jax.experimental.pallas import tpu_sc as plsc

N = 10000
N_PAD = 10240
E = 320000
E_PAD = 327680
CHUNK = 128
D1 = 128
DH = 64
C = 40
D2 = 48
NC = 2
NS = 16
NW = NC * NS
NCHUNKS = E_PAD // CHUNK
ROWS_PER_TILE = N_PAD // NS


def _linear_split_block(x_ref, w_ref, b_ref, o_ref):
    res = lax.dot_general(
        x_ref[...], w_ref[...], (((1,), (1,)), ((), ())),
        preferred_element_type=jnp.float32) + b_ref[...]
    res = res.astype(jnp.bfloat16)
    o_ref[0] = res[:, :DH]
    o_ref[1] = res[:, DH:]


def _linear_split(x, w, b, bm=512):
    m, k = x.shape
    dout = w.shape[0]
    return pl.pallas_call(
        _linear_split_block,
        grid=(m // bm,),
        in_specs=[
            pl.BlockSpec((bm, k), lambda i: (i, 0)),
            pl.BlockSpec((dout, k), lambda i: (0, 0)),
            pl.BlockSpec((1, dout), lambda i: (0, 0)),
        ],
        out_specs=pl.BlockSpec((NC, bm, DH), lambda i: (0, i, 0)),
        out_shape=jax.ShapeDtypeStruct((NC, m, DH), jnp.bfloat16),
    )(x, w, b.reshape(1, dout))


def _fused_linear_block(pa_ref, pb_ref, wa_ref, wb_ref, b_ref, o_ref):
    ha = jnp.maximum(pa_ref[0], 0.0)
    hb = jnp.maximum(pb_ref[0], 0.0)
    dn = (((1,), (1,)), ((), ()))
    o_ref[...] = (
        lax.dot_general(ha, wa_ref[...], dn, preferred_element_type=jnp.float32)
        + lax.dot_general(hb, wb_ref[...], dn, preferred_element_type=jnp.float32)
        + b_ref[...])


def _fused_linear(p, wa, wb, b, bm=512):
    _, m, k = p.shape
    dout = wa.shape[0]
    return pl.pallas_call(
        _fused_linear_block,
        grid=(m // bm,),
        in_specs=[
            pl.BlockSpec((1, bm, k), lambda i: (0, i, 0)),
            pl.BlockSpec((1, bm, k), lambda i: (1, i, 0)),
            pl.BlockSpec((dout, k), lambda i: (0, 0)),
            pl.BlockSpec((dout, k), lambda i: (0, 0)),
            pl.BlockSpec((1, dout), lambda i: (0, 0)),
        ],
        out_specs=pl.BlockSpec((bm, dout), lambda i: (i, 0)),
        out_shape=jax.ShapeDtypeStruct((m, dout), jnp.float32),
    )(p, p, wa, wb, b.reshape(1, dout))


def _logsoftmax_block(pa_ref, pb_ref, o_ref):
    z = pa_ref[0] + pb_ref[0]
    colid = lax.broadcasted_iota(jnp.int32, z.shape, 1)
    valid = colid < C
    zm = jnp.where(valid, z, -jnp.inf)
    mx = jnp.max(zm, axis=1, keepdims=True)
    ex = jnp.where(valid, jnp.exp(z - mx), 0.0)
    s = jnp.sum(ex, axis=1, keepdims=True)
    o_ref[...] = (z - mx - jnp.log(s))[:, :C]


def _logsoftmax(p, bm=400):
    k = p.shape[2]
    return pl.pallas_call(
        _logsoftmax_block,
        grid=(N // bm,),
        in_specs=[
            pl.BlockSpec((1, bm, k), lambda i: (0, i, 0)),
            pl.BlockSpec((1, bm, k), lambda i: (1, i, 0)),
        ],
        out_specs=pl.BlockSpec((bm, C), lambda i: (i, 0)),
        out_shape=jax.ShapeDtypeStruct((N, C), jnp.float32),
    )(p, p)


def _make_agg(d, feature_split, bf16_table):
    nvec = d // 16
    sup_dtype = jnp.bfloat16 if bf16_table else jnp.float32
    mesh = plsc.VectorSubcoreMesh(core_axis_name="c", subcore_axis_name="s")
    nch = NCHUNKS // NS if feature_split else NCHUNKS // NW

    def body(sup_hbm, ei_hbm, w_hbm, zero_hbm, out_hbm,
             col0, col1, row0, row1, w_all, rbuf0, rbuf1, msg0, msg1,
             sup_sh, acc_sh,
             sem_i0, sem_i1, sem_g0, sem_g1, sem_s0, sem_s1):
        cid = lax.axis_index("c")
        sid = lax.axis_index("s")
        slab = pl.ds(sid * ROWS_PER_TILE, ROWS_PER_TILE)
        pltpu.sync_copy(zero_hbm.at[slab], acc_sh.at[slab])
        if feature_split:
            pltpu.sync_copy(sup_hbm.at[cid, slab], sup_sh.at[slab])
            cbase = sid * nch
        else:
            pltpu.sync_copy(sup_hbm.at[slab], sup_sh.at[slab])
            cbase = (sid * NC + cid) * nch
        pltpu.sync_copy(w_hbm.at[pl.ds(cbase * CHUNK, nch * CHUNK)], w_all)

        rbufs = (rbuf0, rbuf1)
        msgs = (msg0, msg1)
        cols = (col0, col1)
        rws = (row0, row1)
        sem_i = (sem_i0, sem_i1)
        sem_g = (sem_g0, sem_g1)
        sem_s = (sem_s0, sem_s1)

        def issue_idx(g, b):
            sl = pl.ds((cbase + g) * CHUNK, CHUNK)
            pltpu.async_copy(ei_hbm.at[pl.ds(1, 1), sl], cols[b], sem_i[b])
            pltpu.async_copy(ei_hbm.at[pl.ds(0, 1), sl], rws[b], sem_i[b])

        def wait_idx(g, b):
            sl = pl.ds((cbase + g) * CHUNK, CHUNK)
            pltpu.make_async_copy(ei_hbm.at[pl.ds(1, 1), sl], cols[b],
                                  sem_i[b]).wait()
            pltpu.make_async_copy(ei_hbm.at[pl.ds(0, 1), sl], rws[b],
                                  sem_i[b]).wait()

        plsc.subcore_barrier()
        issue_idx(0, 0)
        wait_idx(0, 0)
        pltpu.async_copy(sup_sh.at[cols[0].at[0]], rbuf0, sem_g0)
        issue_idx(1, 1)

        def bcast(w16, i):
            return lax.gather(
                w16, jnp.full((16, 1), i, jnp.int32),
                lax.GatherDimensionNumbers(
                    offset_dims=(), collapsed_slice_dims=(0,),
                    start_index_map=(0,)),
                (1,), mode=lax.GatherScatterMode.PROMISE_IN_BOUNDS)

        def scale_chunk(g, rbuf, msg):
            wbase = g * CHUNK
            for i0 in range(CHUNK // 16):
                w16 = w_all[pl.ds(wbase + i0 * 16, 16)]
                for i in range(16):
                    wb = bcast(w16, i)
                    e = i0 * 16 + i
                    if bf16_table:
                        for j2 in range(d // 32):
                            v32 = rbuf[e, pl.ds(j2 * 32, 32)]
                            a, b2 = plsc.unpack(
                                v32, format=plsc.PackFormat.INTERLEAVED)
                            msg[e, pl.ds(j2 * 32, 16)] = a * wb
                            msg[e, pl.ds(j2 * 32 + 16, 16)] = b2 * wb
                    else:
                        for j in range(nvec):
                            fs = pl.ds(j * 16, 16)
                            msg[e, fs] = rbuf[e, fs] * wb

        def step(g, b):
            rbuf, msg = rbufs[b], msgs[b]
            pltpu.make_async_copy(sup_sh.at[cols[b].at[0]], rbuf,
                                  sem_g[b]).wait()

            @pl.when(g >= 2)
            def _():
                pltpu.make_async_copy(msg, acc_sh.at[rws[b].at[0]],
                                      sem_s[b]).wait()

            scale_chunk(g, rbuf, msg)
            pltpu.async_copy(msg, acc_sh.at[rws[b].at[0]], sem_s[b], add=True)

            @pl.when(g + 1 < nch)
            def _():
                wait_idx(g + 1, 1 - b)
                pltpu.async_copy(sup_sh.at[cols[1 - b].at[0]], rbufs[1 - b],
                                 sem_g[1 - b])

            @pl.when(g + 2 < nch)
            def _():
                issue_idx(g + 2, b)

        def outer(g2, c):
            step(g2 * 2, 0)
            step(g2 * 2 + 1, 1)
            return c

        lax.fori_loop(0, nch // 2, outer, 0)
        pltpu.make_async_copy(msg0, acc_sh.at[rws[0].at[0]], sem_s0).wait()
        pltpu.make_async_copy(msg1, acc_sh.at[rws[1].at[0]], sem_s1).wait()
        plsc.subcore_barrier()
        pltpu.sync_copy(acc_sh.at[slab], out_hbm.at[cid, slab])

    return pl.kernel(
        body,
        out_type=jax.ShapeDtypeStruct((NC, N_PAD, d), jnp.float32),
        mesh=mesh,
        compiler_params=pltpu.CompilerParams(
            needs_layout_passes=False, use_tc_tiling_on_sc=False),
        scratch_types=[
            pltpu.VMEM((1, CHUNK), jnp.int32),
            pltpu.VMEM((1, CHUNK), jnp.int32),
            pltpu.VMEM((1, CHUNK), jnp.int32),
            pltpu.VMEM((1, CHUNK), jnp.int32),
            pltpu.VMEM((nch * CHUNK,), jnp.float32),
            pltpu.VMEM((CHUNK, d), sup_dtype),
            pltpu.VMEM((CHUNK, d), sup_dtype),
            pltpu.VMEM((CHUNK, d), jnp.float32),
            pltpu.VMEM((CHUNK, d), jnp.float32),
            pltpu.VMEM_SHARED((N_PAD, d), sup_dtype),
            pltpu.VMEM_SHARED((N_PAD, d), jnp.float32),
            pltpu.SemaphoreType.DMA,
            pltpu.SemaphoreType.DMA,
            pltpu.SemaphoreType.DMA,
            pltpu.SemaphoreType.DMA,
            pltpu.SemaphoreType.DMA,
            pltpu.SemaphoreType.DMA,
        ],
    )


_agg_l1 = _make_agg(DH, feature_split=True, bf16_table=True)
_agg_l2 = _make_agg(D2, feature_split=False, bf16_table=False)

_BF16_PERM = tuple(
    32 * (m // 32) + 2 * (m % 16) + (m % 32) // 16 for m in range(DH))


@jax.jit
def _run(node_features, edge_index, edge_weight, W1, b1, W2, b2):
    ei = jnp.pad(edge_index.astype(jnp.int32), ((0, 0), (0, E_PAD - E)))
    x = jnp.pad(node_features, ((0, N_PAD - N), (0, 0)))
    wp = jnp.pad(edge_weight.astype(jnp.float32), (0, E_PAD - E))
    w2p = jnp.pad(W2, ((0, D2 - C), (0, 0)))
    b2p = jnp.pad(b2, (0, D2 - C))
    perm = jnp.array(_BF16_PERM, dtype=jnp.int32)
    w2a = w2p[:, :DH][:, perm]
    w2b = w2p[:, DH:][:, perm]
    zeros1 = jnp.zeros((N_PAD, DH), jnp.float32)
    zeros2 = jnp.zeros((N_PAD, D2), jnp.float32)

    sup1 = _linear_split(x, W1, b1)
    p1 = _agg_l1(sup1, ei, wp, zeros1)
    sup2 = _fused_linear(p1, w2a, w2b, b2p)
    p2 = _agg_l2(sup2, ei, wp, zeros2)
    return _logsoftmax(p2)


def kernel(node_features, edge_index, edge_weight, W1, b1, W2, b2):
    return _run(node_features, edge_index, edge_weight, W1, b1, W2, b2)


# speedup vs baseline: 10.7336x; 1.0810x over previous
import functools

import jax
import jax.numpy as jnp
from jax import lax
from jax.experimental import pallas as pl
from jax.experimental.pallas import tpu as pltpu
from jax.experimental.pallas import tpu_sc as plsc

N = 10000
N_PAD = 10240
E = 320000
E_PAD = 327680
CHUNK = 128
D1 = 128
DH = 64
C = 40
D2 = 48
NC = 2
NS = 16
NW = NC * NS
NCHUNKS = E_PAD // CHUNK
ROWS_PER_TILE = N_PAD // NS


def _linear_split_block(x_ref, w_ref, b_ref, o_ref):
    res = lax.dot_general(
        x_ref[...], w_ref[...], (((1,), (1,)), ((), ())),
        preferred_element_type=jnp.float32) + b_ref[...]
    res = res.astype(jnp.bfloat16)
    o_ref[0] = res[:, :DH]
    o_ref[1] = res[:, DH:]


def _linear_split(x, w, b, bm=2048):
    m, k = x.shape
    dout = w.shape[0]
    return pl.pallas_call(
        _linear_split_block,
        grid=(m // bm,),
        in_specs=[
            pl.BlockSpec((bm, k), lambda i: (i, 0)),
            pl.BlockSpec((dout, k), lambda i: (0, 0)),
            pl.BlockSpec((1, dout), lambda i: (0, 0)),
        ],
        out_specs=pl.BlockSpec((NC, bm, DH), lambda i: (0, i, 0)),
        out_shape=jax.ShapeDtypeStruct((NC, m, DH), jnp.bfloat16),
    )(x, w, b.reshape(1, dout))


def _fused_linear_block(pa_ref, pb_ref, wa_ref, wb_ref, b_ref, o_ref):
    ha = jnp.maximum(pa_ref[0], 0.0)
    hb = jnp.maximum(pb_ref[0], 0.0)
    dn = (((1,), (1,)), ((), ()))
    o_ref[...] = (
        lax.dot_general(ha, wa_ref[...], dn, preferred_element_type=jnp.float32)
        + lax.dot_general(hb, wb_ref[...], dn, preferred_element_type=jnp.float32)
        + b_ref[...])


def _fused_linear(p, wa, wb, b, bm=2048):
    _, m, k = p.shape
    dout = wa.shape[0]
    return pl.pallas_call(
        _fused_linear_block,
        grid=(m // bm,),
        in_specs=[
            pl.BlockSpec((1, bm, k), lambda i: (0, i, 0)),
            pl.BlockSpec((1, bm, k), lambda i: (1, i, 0)),
            pl.BlockSpec((dout, k), lambda i: (0, 0)),
            pl.BlockSpec((dout, k), lambda i: (0, 0)),
            pl.BlockSpec((1, dout), lambda i: (0, 0)),
        ],
        out_specs=pl.BlockSpec((bm, dout), lambda i: (i, 0)),
        out_shape=jax.ShapeDtypeStruct((m, dout), jnp.float32),
    )(p, p, wa, wb, b.reshape(1, dout))


def _logsoftmax_block(pa_ref, pb_ref, o_ref):
    z = pa_ref[0] + pb_ref[0]
    colid = lax.broadcasted_iota(jnp.int32, z.shape, 1)
    valid = colid < C
    zm = jnp.where(valid, z, -jnp.inf)
    mx = jnp.max(zm, axis=1, keepdims=True)
    ex = jnp.where(valid, jnp.exp(z - mx), 0.0)
    s = jnp.sum(ex, axis=1, keepdims=True)
    o_ref[...] = (z - mx - jnp.log(s))[:, :C]


def _logsoftmax(p, bm=2000):
    k = p.shape[2]
    return pl.pallas_call(
        _logsoftmax_block,
        grid=(N // bm,),
        in_specs=[
            pl.BlockSpec((1, bm, k), lambda i: (0, i, 0)),
            pl.BlockSpec((1, bm, k), lambda i: (1, i, 0)),
        ],
        out_specs=pl.BlockSpec((bm, C), lambda i: (i, 0)),
        out_shape=jax.ShapeDtypeStruct((N, C), jnp.float32),
    )(p, p)


def _make_agg(d, feature_split, bf16_table):
    nvec = d // 16
    sup_dtype = jnp.bfloat16 if bf16_table else jnp.float32
    mesh = plsc.VectorSubcoreMesh(core_axis_name="c", subcore_axis_name="s")
    nch = NCHUNKS // NS if feature_split else NCHUNKS // NW

    def body(sup_hbm, ei_hbm, w_hbm, zero_hbm, out_hbm,
             col0, col1, row0, row1, w_all, rbuf0, rbuf1, msg0, msg1,
             sup_sh, acc_sh,
             sem_i0, sem_i1, sem_g0, sem_g1, sem_s0, sem_s1):
        cid = lax.axis_index("c")
        sid = lax.axis_index("s")
        slab = pl.ds(sid * ROWS_PER_TILE, ROWS_PER_TILE)
        pltpu.sync_copy(zero_hbm.at[slab], acc_sh.at[slab])
        if feature_split:
            pltpu.sync_copy(sup_hbm.at[cid, slab], sup_sh.at[slab])
            cbase = sid * nch
        else:
            pltpu.sync_copy(sup_hbm.at[slab], sup_sh.at[slab])
            cbase = (sid * NC + cid) * nch
        pltpu.sync_copy(w_hbm.at[pl.ds(cbase * CHUNK, nch * CHUNK)], w_all)

        rbufs = (rbuf0, rbuf1)
        msgs = (msg0, msg1)
        cols = (col0, col1)
        rws = (row0, row1)
        sem_i = (sem_i0, sem_i1)
        sem_g = (sem_g0, sem_g1)
        sem_s = (sem_s0, sem_s1)

        def issue_idx(g, b):
            sl = pl.ds((cbase + g) * CHUNK, CHUNK)
            pltpu.async_copy(ei_hbm.at[pl.ds(1, 1), sl], cols[b], sem_i[b])
            pltpu.async_copy(ei_hbm.at[pl.ds(0, 1), sl], rws[b], sem_i[b])

        def wait_idx(g, b):
            sl = pl.ds((cbase + g) * CHUNK, CHUNK)
            pltpu.make_async_copy(ei_hbm.at[pl.ds(1, 1), sl], cols[b],
                                  sem_i[b]).wait()
            pltpu.make_async_copy(ei_hbm.at[pl.ds(0, 1), sl], rws[b],
                                  sem_i[b]).wait()

        plsc.subcore_barrier()
        issue_idx(0, 0)
        wait_idx(0, 0)
        pltpu.async_copy(sup_sh.at[cols[0].at[0]], rbuf0, sem_g0)
        issue_idx(1, 1)

        def bcast(w16, i):
            return lax.gather(
                w16, jnp.full((16, 1), i, jnp.int32),
                lax.GatherDimensionNumbers(
                    offset_dims=(), collapsed_slice_dims=(0,),
                    start_index_map=(0,)),
                (1,), mode=lax.GatherScatterMode.PROMISE_IN_BOUNDS)

        def scale_chunk(g, rbuf, msg):
            wbase = g * CHUNK
            for i0 in range(CHUNK // 16):
                w16 = w_all[pl.ds(wbase + i0 * 16, 16)]
                for i in range(16):
                    wb = bcast(w16, i)
                    e = i0 * 16 + i
                    if bf16_table:
                        for j2 in range(d // 32):
                            v32 = rbuf[e, pl.ds(j2 * 32, 32)]
                            a, b2 = plsc.unpack(
                                v32, format=plsc.PackFormat.INTERLEAVED)
                            msg[e, pl.ds(j2 * 32, 16)] = a * wb
                            msg[e, pl.ds(j2 * 32 + 16, 16)] = b2 * wb
                    else:
                        for j in range(nvec):
                            fs = pl.ds(j * 16, 16)
                            msg[e, fs] = rbuf[e, fs] * wb

        def step(g, b):
            rbuf, msg = rbufs[b], msgs[b]
            pltpu.make_async_copy(sup_sh.at[cols[b].at[0]], rbuf,
                                  sem_g[b]).wait()

            @pl.when(g >= 2)
            def _():
                pltpu.make_async_copy(msg, acc_sh.at[rws[b].at[0]],
                                      sem_s[b]).wait()

            scale_chunk(g, rbuf, msg)
            pltpu.async_copy(msg, acc_sh.at[rws[b].at[0]], sem_s[b], add=True)

            @pl.when(g + 1 < nch)
            def _():
                wait_idx(g + 1, 1 - b)
                pltpu.async_copy(sup_sh.at[cols[1 - b].at[0]], rbufs[1 - b],
                                 sem_g[1 - b])

            @pl.when(g + 2 < nch)
            def _():
                issue_idx(g + 2, b)

        def outer(g2, c):
            step(g2 * 2, 0)
            step(g2 * 2 + 1, 1)
            return c

        lax.fori_loop(0, nch // 2, outer, 0)
        pltpu.make_async_copy(msg0, acc_sh.at[rws[0].at[0]], sem_s0).wait()
        pltpu.make_async_copy(msg1, acc_sh.at[rws[1].at[0]], sem_s1).wait()
        plsc.subcore_barrier()
        pltpu.sync_copy(acc_sh.at[slab], out_hbm.at[cid, slab])

    return pl.kernel(
        body,
        out_type=jax.ShapeDtypeStruct((NC, N_PAD, d), jnp.float32),
        mesh=mesh,
        compiler_params=pltpu.CompilerParams(
            needs_layout_passes=False, use_tc_tiling_on_sc=False),
        scratch_types=[
            pltpu.VMEM((1, CHUNK), jnp.int32),
            pltpu.VMEM((1, CHUNK), jnp.int32),
            pltpu.VMEM((1, CHUNK), jnp.int32),
            pltpu.VMEM((1, CHUNK), jnp.int32),
            pltpu.VMEM((nch * CHUNK,), jnp.float32),
            pltpu.VMEM((CHUNK, d), sup_dtype),
            pltpu.VMEM((CHUNK, d), sup_dtype),
            pltpu.VMEM((CHUNK, d), jnp.float32),
            pltpu.VMEM((CHUNK, d), jnp.float32),
            pltpu.VMEM_SHARED((N_PAD, d), sup_dtype),
            pltpu.VMEM_SHARED((N_PAD, d), jnp.float32),
            pltpu.SemaphoreType.DMA,
            pltpu.SemaphoreType.DMA,
            pltpu.SemaphoreType.DMA,
            pltpu.SemaphoreType.DMA,
            pltpu.SemaphoreType.DMA,
            pltpu.SemaphoreType.DMA,
        ],
    )


_agg_l1 = _make_agg(DH, feature_split=True, bf16_table=True)
_agg_l2 = _make_agg(D2, feature_split=False, bf16_table=False)

_BF16_PERM = tuple(
    32 * (m // 32) + 2 * (m % 16) + (m % 32) // 16 for m in range(DH))


@jax.jit
def _run(node_features, edge_index, edge_weight, W1, b1, W2, b2):
    ei = jnp.pad(edge_index.astype(jnp.int32), ((0, 0), (0, E_PAD - E)))
    x = jnp.pad(node_features, ((0, N_PAD - N), (0, 0)))
    wp = jnp.pad(edge_weight.astype(jnp.float32), (0, E_PAD - E))
    w2p = jnp.pad(W2, ((0, D2 - C), (0, 0)))
    b2p = jnp.pad(b2, (0, D2 - C))
    perm = jnp.array(_BF16_PERM, dtype=jnp.int32)
    w2a = w2p[:, :DH][:, perm]
    w2b = w2p[:, DH:][:, perm]
    zeros1 = jnp.zeros((N_PAD, DH), jnp.float32)
    zeros2 = jnp.zeros((N_PAD, D2), jnp.float32)

    sup1 = _linear_split(x, W1, b1)
    p1 = _agg_l1(sup1, ei, wp, zeros1)
    sup2 = _fused_linear(p1, w2a, w2b, b2p)
    p2 = _agg_l2(sup2, ei, wp, zeros2)
    return _logsoftmax(p2)


def kernel(node_features, edge_index, edge_weight, W1, b1, W2, b2):
    return _run(node_features, edge_index, edge_weight, W1, b1, W2, b2)


# speedup vs baseline: 10.7369x; 1.0003x over previous
import jax
import jax.numpy as jnp
from jax import lax
from jax.experimental import pallas as pl
from jax.experimental.pallas import tpu as pltpu
from jax.experimental.pallas import tpu_sc as plsc

N = 10000
N_PAD = 10240
E = 320000
E_PAD = 327680
CHUNK = 128
D1 = 128
DH = 64
C = 40
D2 = 48
NC = 2
NS = 16
NW = NC * NS
NCHUNKS = E_PAD // CHUNK
ROWS_PER_TILE = N_PAD // NS


def _linear_split_block(x_ref, w_ref, b_ref, o_ref):
    res = lax.dot_general(
        x_ref[...], w_ref[...], (((1,), (1,)), ((), ())),
        preferred_element_type=jnp.float32) + b_ref[...]
    res = res.astype(jnp.bfloat16)
    o_ref[0] = res[:, :DH]
    o_ref[1] = res[:, DH:]


def _linear_split(x, w, b, bm=2048):
    m, k = x.shape
    dout = w.shape[0]
    return pl.pallas_call(
        _linear_split_block,
        grid=(m // bm,),
        in_specs=[
            pl.BlockSpec((bm, k), lambda i: (i, 0)),
            pl.BlockSpec((dout, k), lambda i: (0, 0)),
            pl.BlockSpec((1, dout), lambda i: (0, 0)),
        ],
        out_specs=pl.BlockSpec((NC, bm, DH), lambda i: (0, i, 0)),
        out_shape=jax.ShapeDtypeStruct((NC, m, DH), jnp.bfloat16),
    )(x, w, b.reshape(1, dout))


def _fused_linear_block(pa_ref, pb_ref, wa_ref, wb_ref, b_ref, o_ref):
    ha = jnp.maximum(pa_ref[0], 0.0)
    hb = jnp.maximum(pb_ref[0], 0.0)
    dn = (((1,), (1,)), ((), ()))
    o_ref[...] = (
        lax.dot_general(ha, wa_ref[...], dn, preferred_element_type=jnp.float32)
        + lax.dot_general(hb, wb_ref[...], dn, preferred_element_type=jnp.float32)
        + b_ref[...])


def _fused_linear(p, wa, wb, b, bm=2048):
    _, m, k = p.shape
    dout = wa.shape[0]
    return pl.pallas_call(
        _fused_linear_block,
        grid=(m // bm,),
        in_specs=[
            pl.BlockSpec((1, bm, k), lambda i: (0, i, 0)),
            pl.BlockSpec((1, bm, k), lambda i: (1, i, 0)),
            pl.BlockSpec((dout, k), lambda i: (0, 0)),
            pl.BlockSpec((dout, k), lambda i: (0, 0)),
            pl.BlockSpec((1, dout), lambda i: (0, 0)),
        ],
        out_specs=pl.BlockSpec((bm, dout), lambda i: (i, 0)),
        out_shape=jax.ShapeDtypeStruct((m, dout), jnp.float32),
    )(p, p, wa, wb, b.reshape(1, dout))


def _logsoftmax_block(pa_ref, pb_ref, o_ref):
    z = pa_ref[0] + pb_ref[0]
    colid = lax.broadcasted_iota(jnp.int32, z.shape, 1)
    valid = colid < C
    zm = jnp.where(valid, z, -jnp.inf)
    mx = jnp.max(zm, axis=1, keepdims=True)
    ex = jnp.where(valid, jnp.exp(z - mx), 0.0)
    s = jnp.sum(ex, axis=1, keepdims=True)
    o_ref[...] = (z - mx - jnp.log(s))[:, :C]


def _logsoftmax(p, bm=2000):
    k = p.shape[2]
    return pl.pallas_call(
        _logsoftmax_block,
        grid=(N // bm,),
        in_specs=[
            pl.BlockSpec((1, bm, k), lambda i: (0, i, 0)),
            pl.BlockSpec((1, bm, k), lambda i: (1, i, 0)),
        ],
        out_specs=pl.BlockSpec((bm, C), lambda i: (i, 0)),
        out_shape=jax.ShapeDtypeStruct((N, C), jnp.float32),
    )(p, p)


def _make_agg(d, feature_split, bf16_table):
    nvec = d // 16
    sup_dtype = jnp.bfloat16 if bf16_table else jnp.float32
    mesh = plsc.VectorSubcoreMesh(core_axis_name="c", subcore_axis_name="s")
    nch = NCHUNKS // NS if feature_split else NCHUNKS // NW

    def body(sup_hbm, ei_hbm, w_hbm, zero_hbm, out_hbm,
             col0, col1, row0, row1, w_all, rbuf0, rbuf1, msg0, msg1,
             sup_sh, acc_sh,
             sem_i0, sem_i1, sem_g0, sem_g1, sem_s0, sem_s1):
        cid = lax.axis_index("c")
        sid = lax.axis_index("s")
        slab = pl.ds(sid * ROWS_PER_TILE, ROWS_PER_TILE)
        pltpu.sync_copy(zero_hbm.at[slab], acc_sh.at[slab])
        if feature_split:
            pltpu.sync_copy(sup_hbm.at[cid, slab], sup_sh.at[slab])
            cbase = sid * nch
        else:
            pltpu.sync_copy(sup_hbm.at[slab], sup_sh.at[slab])
            cbase = (sid * NC + cid) * nch
        pltpu.sync_copy(w_hbm.at[pl.ds(cbase * CHUNK, nch * CHUNK)], w_all)

        rbufs = (rbuf0, rbuf1)
        msgs = (msg0, msg1)
        cols = (col0, col1)
        rws = (row0, row1)
        sem_i = (sem_i0, sem_i1)
        sem_g = (sem_g0, sem_g1)
        sem_s = (sem_s0, sem_s1)

        def issue_idx(g, b):
            sl = pl.ds((cbase + g) * CHUNK, CHUNK)
            pltpu.async_copy(ei_hbm.at[pl.ds(1, 1), sl], cols[b], sem_i[b])
            pltpu.async_copy(ei_hbm.at[pl.ds(0, 1), sl], rws[b], sem_i[b])

        def wait_idx(g, b):
            sl = pl.ds((cbase + g) * CHUNK, CHUNK)
            pltpu.make_async_copy(ei_hbm.at[pl.ds(1, 1), sl], cols[b],
                                  sem_i[b]).wait()
            pltpu.make_async_copy(ei_hbm.at[pl.ds(0, 1), sl], rws[b],
                                  sem_i[b]).wait()

        plsc.subcore_barrier()
        issue_idx(0, 0)
        wait_idx(0, 0)
        pltpu.async_copy(sup_sh.at[cols[0].at[0]], rbuf0, sem_g0)
        issue_idx(1, 1)

        def bcast(w16, i):
            return lax.gather(
                w16, jnp.full((16, 1), i, jnp.int32),
                lax.GatherDimensionNumbers(
                    offset_dims=(), collapsed_slice_dims=(0,),
                    start_index_map=(0,)),
                (1,), mode=lax.GatherScatterMode.PROMISE_IN_BOUNDS)

        def scale_chunk(g, rbuf, msg):
            wbase = g * CHUNK
            for i0 in range(CHUNK // 16):
                w16 = w_all[pl.ds(wbase + i0 * 16, 16)]
                for i in range(16):
                    wb = bcast(w16, i)
                    e = i0 * 16 + i
                    if bf16_table:
                        for j2 in range(d // 32):
                            v32 = rbuf[e, pl.ds(j2 * 32, 32)]
                            a, b2 = plsc.unpack(
                                v32, format=plsc.PackFormat.INTERLEAVED)
                            msg[e, pl.ds(j2 * 32, 16)] = a * wb
                            msg[e, pl.ds(j2 * 32 + 16, 16)] = b2 * wb
                    else:
                        for j in range(nvec):
                            fs = pl.ds(j * 16, 16)
                            msg[e, fs] = rbuf[e, fs] * wb

        def step(g, b):
            rbuf, msg = rbufs[b], msgs[b]
            pltpu.make_async_copy(sup_sh.at[cols[b].at[0]], rbuf,
                                  sem_g[b]).wait()

            @pl.when(g >= 2)
            def _():
                pltpu.make_async_copy(msg, acc_sh.at[rws[b].at[0]],
                                      sem_s[b]).wait()

            scale_chunk(g, rbuf, msg)
            pltpu.async_copy(msg, acc_sh.at[rws[b].at[0]], sem_s[b], add=True)

            @pl.when(g + 1 < nch)
            def _():
                wait_idx(g + 1, 1 - b)
                pltpu.async_copy(sup_sh.at[cols[1 - b].at[0]], rbufs[1 - b],
                                 sem_g[1 - b])

            @pl.when(g + 2 < nch)
            def _():
                issue_idx(g + 2, b)

        def outer(g2, c):
            step(g2 * 2, 0)
            step(g2 * 2 + 1, 1)
            return c

        lax.fori_loop(0, nch // 2, outer, 0)
        pltpu.make_async_copy(msg0, acc_sh.at[rws[0].at[0]], sem_s0).wait()
        pltpu.make_async_copy(msg1, acc_sh.at[rws[1].at[0]], sem_s1).wait()
        plsc.subcore_barrier()
        pltpu.sync_copy(acc_sh.at[slab], out_hbm.at[cid, slab])

    return pl.kernel(
        body,
        out_type=jax.ShapeDtypeStruct((NC, N_PAD, d), jnp.float32),
        mesh=mesh,
        compiler_params=pltpu.CompilerParams(
            needs_layout_passes=False, use_tc_tiling_on_sc=False),
        scratch_types=[
            pltpu.VMEM((1, CHUNK), jnp.int32),
            pltpu.VMEM((1, CHUNK), jnp.int32),
            pltpu.VMEM((1, CHUNK), jnp.int32),
            pltpu.VMEM((1, CHUNK), jnp.int32),
            pltpu.VMEM((nch * CHUNK,), jnp.float32),
            pltpu.VMEM((CHUNK, d), sup_dtype),
            pltpu.VMEM((CHUNK, d), sup_dtype),
            pltpu.VMEM((CHUNK, d), jnp.float32),
            pltpu.VMEM((CHUNK, d), jnp.float32),
            pltpu.VMEM_SHARED((N_PAD, d), sup_dtype),
            pltpu.VMEM_SHARED((N_PAD, d), jnp.float32),
            pltpu.SemaphoreType.DMA,
            pltpu.SemaphoreType.DMA,
            pltpu.SemaphoreType.DMA,
            pltpu.SemaphoreType.DMA,
            pltpu.SemaphoreType.DMA,
            pltpu.SemaphoreType.DMA,
        ],
    )


_agg_l1 = _make_agg(DH, feature_split=True, bf16_table=True)
_agg_l2 = _make_agg(D2, feature_split=False, bf16_table=False)

_BF16_PERM = tuple(
    32 * (m // 32) + 2 * (m % 16) + (m % 32) // 16 for m in range(DH))


@jax.jit
def _run(node_features, edge_index, edge_weight, W1, b1, W2, b2):
    ei = jnp.pad(edge_index.astype(jnp.int32), ((0, 0), (0, E_PAD - E)))
    x = jnp.pad(node_features, ((0, N_PAD - N), (0, 0)))
    wp = jnp.pad(edge_weight.astype(jnp.float32), (0, E_PAD - E))
    w2p = jnp.pad(W2, ((0, D2 - C), (0, 0)))
    b2p = jnp.pad(b2, (0, D2 - C))
    perm = jnp.array(_BF16_PERM, dtype=jnp.int32)
    w2a = w2p[:, :DH][:, perm]
    w2b = w2p[:, DH:][:, perm]
    zeros1 = jnp.zeros((N_PAD, DH), jnp.float32)
    zeros2 = jnp.zeros((N_PAD, D2), jnp.float32)

    sup1 = _linear_split(x, W1, b1)
    p1 = _agg_l1(sup1, ei, wp, zeros1)
    sup2 = _fused_linear(p1, w2a, w2b, b2p)
    p2 = _agg_l2(sup2, ei, wp, zeros2)
    return _logsoftmax(p2)


def kernel(node_features, edge_index, edge_weight, W1, b1, W2, b2):
    return _run(node_features, edge_index, edge_weight, W1, b1, W2, b2)
